# Initial kernel scaffold; baseline (speedup 1.0000x reference)
import jax
import jax.numpy as jnp
from jax.experimental import pallas as pl


def kernel(x, o_adj, s_adj, idx, Wo1, bo1, Wo2, bo2, Ws1, bs1, Ws2, bs2, ag1, ag2, Wd1, bd1, Wd2, bd2):
    raise NotImplementedError("write your pallas kernel here")


# speedup vs baseline: 2.2518x; 2.2518x over previous
import functools

import jax
import jax.numpy as jnp
from jax import lax
from jax.experimental import pallas as pl
from jax.experimental.pallas import tpu as pltpu
from jax.experimental.pallas import tpu_sc as plsc


def _prep_body(x_ref, wo_ref, ws_ref, so_ref, ss_ref):
    xb = x_ref[...]
    so_ref[...] = jnp.dot(xb, wo_ref[...], preferred_element_type=jnp.float32)
    ss_ref[...] = jnp.dot(xb, ws_ref[...], preferred_element_type=jnp.float32)


def _passA_body(oadj_ref, sadj_ref, so_ref, ss_ref, bo1_ref, bs1_ref,
                wo2_ref, ws2_ref, to_ref, ts_ref):
    h_o = jnp.dot(oadj_ref[...], so_ref[...], preferred_element_type=jnp.float32)
    h_o = jnp.maximum(h_o + bo1_ref[...], 0.0)
    to_ref[...] = jnp.dot(h_o, wo2_ref[...], preferred_element_type=jnp.float32)
    h_s = jnp.dot(sadj_ref[...], ss_ref[...], preferred_element_type=jnp.float32)
    h_s = jnp.maximum(h_s + bs1_ref[...], 0.0)
    ts_ref[...] = jnp.dot(h_s, ws2_ref[...], preferred_element_type=jnp.float32)


def _passB_body(oadj_ref, sadj_ref, to_ref, ts_ref, bo2_ref, bs2_ref,
                ag1_ref, ag2_ref, wa_ref, wb_ref, c_ref, pq_ref):
    u = jnp.dot(oadj_ref[...], to_ref[...], preferred_element_type=jnp.float32)
    u = u + bo2_ref[...]
    v = jnp.dot(sadj_ref[...], ts_ref[...], preferred_element_type=jnp.float32)
    v = v + bs2_ref[...]
    a_o = jnp.sum(u * ag1_ref[...], axis=1, keepdims=True)
    a_s = jnp.sum(v * ag2_ref[...], axis=1, keepdims=True)
    p = a_o * jnp.sum(u * wa_ref[...], axis=1, keepdims=True) \
        + a_s * jnp.sum(v * wa_ref[...], axis=1, keepdims=True) + c_ref[...]
    q = a_o * jnp.sum(u * wb_ref[...], axis=1, keepdims=True) \
        + a_s * jnp.sum(v * wb_ref[...], axis=1, keepdims=True)
    bm = p.shape[0]
    pq_ref[...] = jnp.concatenate(
        [p, q, jnp.zeros((bm, 6), jnp.float32)], axis=1)


def _edge_gather(p, q, idx0, idx1, n_workers, nc):
    (e_total,) = idx0.shape
    n_nodes = p.shape[0]
    ew = e_total // n_workers
    steps = (ew + 15) // 16
    pad = steps * 16

    mesh = plsc.VectorSubcoreMesh(core_axis_name="c", subcore_axis_name="s")

    @functools.partial(
        pl.kernel,
        mesh=mesh,
        compiler_params=pltpu.CompilerParams(needs_layout_passes=False),
        out_type=jax.ShapeDtypeStruct((e_total,), jnp.float32),
        scratch_types=[
            pltpu.VMEM((n_nodes,), jnp.float32),
            pltpu.VMEM((n_nodes,), jnp.float32),
            pltpu.VMEM((pad,), jnp.int32),
            pltpu.VMEM((pad,), jnp.int32),
            pltpu.VMEM((pad,), jnp.float32),
        ],
    )
    def k(p_hbm, q_hbm, i0_hbm, i1_hbm, out_hbm, p_v, q_v, i0_v, i1_v, out_v):
        wid = lax.axis_index("s") * nc + lax.axis_index("c")
        base = wid * ew
        pltpu.sync_copy(p_hbm, p_v)
        pltpu.sync_copy(q_hbm, q_v)
        if pad > ew:
            zeros16 = jnp.zeros((16,), jnp.int32)
            i0_v[pl.ds(pad - 16, 16)] = zeros16
            i1_v[pl.ds(pad - 16, 16)] = zeros16
        pltpu.sync_copy(i0_hbm.at[pl.ds(base, ew)], i0_v.at[pl.ds(0, ew)])
        pltpu.sync_copy(i1_hbm.at[pl.ds(base, ew)], i1_v.at[pl.ds(0, ew)])

        def body(k_it, _):
            off = k_it * 16
            g0 = plsc.load_gather(p_v, [i0_v[pl.ds(off, 16)]])
            g1 = plsc.load_gather(q_v, [i1_v[pl.ds(off, 16)]])
            out_v[pl.ds(off, 16)] = g0 + g1
            return _

        lax.fori_loop(0, steps, body, None)
        pltpu.sync_copy(out_v.at[pl.ds(0, ew)], out_hbm.at[pl.ds(base, ew)])

    return k(p, q, idx0, idx1)


def kernel(x, o_adj, s_adj, idx, Wo1, bo1, Wo2, bo2, Ws1, bs1, Ws2, bs2,
           ag1, ag2, Wd1, bd1, Wd2, bd2):
    n, nfeat = x.shape
    h1 = Wo1.shape[1]
    h2 = Wo2.shape[1]
    e_total = idx.shape[1]

    w = Wd1 @ Wd2
    wa = w[:h2, 0][None, :]
    wb = w[h2:, 0][None, :]
    c = (bd1 @ Wd2 + bd2).reshape(1, 1)

    bmp = 2000
    s_o, s_s = pl.pallas_call(
        _prep_body,
        grid=(pl.cdiv(n, bmp),),
        in_specs=[
            pl.BlockSpec((bmp, nfeat), lambda i: (i, 0)),
            pl.BlockSpec((nfeat, h1), lambda i: (0, 0)),
            pl.BlockSpec((nfeat, h1), lambda i: (0, 0)),
        ],
        out_specs=[
            pl.BlockSpec((bmp, h1), lambda i: (i, 0)),
            pl.BlockSpec((bmp, h1), lambda i: (i, 0)),
        ],
        out_shape=[
            jax.ShapeDtypeStruct((n, h1), jnp.float32),
            jax.ShapeDtypeStruct((n, h1), jnp.float32),
        ],
    )(x, Wo1, Ws1)

    bm = 256
    grid = (pl.cdiv(n, bm),)
    adj_spec = pl.BlockSpec((bm, n), lambda i: (i, 0))
    full = lambda r, c_: pl.BlockSpec((r, c_), lambda i: (0, 0))

    t_o, t_s = pl.pallas_call(
        _passA_body,
        grid=grid,
        in_specs=[
            adj_spec, adj_spec,
            full(n, h1), full(n, h1),
            full(1, h1), full(1, h1),
            full(h1, h2), full(h1, h2),
        ],
        out_specs=[
            pl.BlockSpec((bm, h2), lambda i: (i, 0)),
            pl.BlockSpec((bm, h2), lambda i: (i, 0)),
        ],
        out_shape=[
            jax.ShapeDtypeStruct((n, h2), jnp.float32),
            jax.ShapeDtypeStruct((n, h2), jnp.float32),
        ],
    )(o_adj, s_adj, s_o, s_s, bo1[None, :], bs1[None, :], Wo2, Ws2)

    pq = pl.pallas_call(
        _passB_body,
        grid=grid,
        in_specs=[
            adj_spec, adj_spec,
            full(n, h2), full(n, h2),
            full(1, h2), full(1, h2),
            full(1, h2), full(1, h2),
            full(1, h2), full(1, h2),
            full(1, 1),
        ],
        out_specs=pl.BlockSpec((bm, 8), lambda i: (i, 0)),
        out_shape=jax.ShapeDtypeStruct((n, 8), jnp.float32),
    )(o_adj, s_adj, t_o, t_s, bo2[None, :], bs2[None, :],
      ag1[None, :], ag2[None, :], wa, wb, c)

    p = pq[:, 0]
    q = pq[:, 1]

    info = plsc.get_sparse_core_info()
    nc, ns = info.num_cores, info.num_subcores
    out = _edge_gather(p, q, idx[0], idx[1], nc * ns, nc)
    return out[:, None]


# speedup vs baseline: 2.3519x; 1.0444x over previous
import functools

import jax
import jax.numpy as jnp
from jax import lax
from jax.experimental import pallas as pl
from jax.experimental.pallas import tpu as pltpu
from jax.experimental.pallas import tpu_sc as plsc


def _prep_body(x_ref, wo_ref, ws_ref, so_ref, ss_ref):
    xb = x_ref[...]
    so_ref[...] = jnp.dot(xb, wo_ref[...], preferred_element_type=jnp.float32)
    ss_ref[...] = jnp.dot(xb, ws_ref[...], preferred_element_type=jnp.float32)


def _passA_body(oadj_ref, sadj_ref, so_ref, ss_ref, bo1_ref, bs1_ref,
                wo2_ref, ws2_ref, to_ref, ts_ref):
    h_o = jnp.dot(oadj_ref[...], so_ref[...], preferred_element_type=jnp.float32)
    h_o = jnp.maximum(h_o + bo1_ref[...], 0.0)
    to_ref[...] = jnp.dot(h_o, wo2_ref[...], preferred_element_type=jnp.float32)
    h_s = jnp.dot(sadj_ref[...], ss_ref[...], preferred_element_type=jnp.float32)
    h_s = jnp.maximum(h_s + bs1_ref[...], 0.0)
    ts_ref[...] = jnp.dot(h_s, ws2_ref[...], preferred_element_type=jnp.float32)


def _passB_body(oadj_ref, sadj_ref, to_ref, ts_ref, bo2_ref, bs2_ref,
                ag1_ref, ag2_ref, wa_ref, wb_ref, c_ref, pq_ref):
    u = jnp.dot(oadj_ref[...], to_ref[...], preferred_element_type=jnp.float32)
    u = u + bo2_ref[...]
    v = jnp.dot(sadj_ref[...], ts_ref[...], preferred_element_type=jnp.float32)
    v = v + bs2_ref[...]
    a_o = jnp.sum(u * ag1_ref[...], axis=1, keepdims=True)
    a_s = jnp.sum(v * ag2_ref[...], axis=1, keepdims=True)
    p = a_o * jnp.sum(u * wa_ref[...], axis=1, keepdims=True) \
        + a_s * jnp.sum(v * wa_ref[...], axis=1, keepdims=True) + c_ref[...]
    q = a_o * jnp.sum(u * wb_ref[...], axis=1, keepdims=True) \
        + a_s * jnp.sum(v * wb_ref[...], axis=1, keepdims=True)
    bm = p.shape[0]
    pq_ref[...] = jnp.concatenate(
        [p, q, jnp.zeros((bm, 6), jnp.float32)], axis=1)


def _edge_gather(p, q, idx0, idx1, n_workers, nc):
    (e_total,) = idx0.shape
    n_nodes = p.shape[0]
    ew = e_total // n_workers
    steps = (ew + 15) // 16
    pad = steps * 16

    mesh = plsc.VectorSubcoreMesh(core_axis_name="c", subcore_axis_name="s")

    @functools.partial(
        pl.kernel,
        mesh=mesh,
        compiler_params=pltpu.CompilerParams(needs_layout_passes=False),
        out_type=jax.ShapeDtypeStruct((e_total,), jnp.float32),
        scratch_types=[
            pltpu.VMEM((n_nodes,), jnp.float32),
            pltpu.VMEM((n_nodes,), jnp.float32),
            pltpu.VMEM((pad,), jnp.int32),
            pltpu.VMEM((pad,), jnp.int32),
            pltpu.VMEM((pad,), jnp.float32),
        ],
    )
    def k(p_hbm, q_hbm, i0_hbm, i1_hbm, out_hbm, p_v, q_v, i0_v, i1_v, out_v):
        wid = lax.axis_index("s") * nc + lax.axis_index("c")
        base = wid * ew
        pltpu.sync_copy(p_hbm, p_v)
        pltpu.sync_copy(q_hbm, q_v)
        if pad > ew:
            zeros16 = jnp.zeros((16,), jnp.int32)
            i0_v[pl.ds(pad - 16, 16)] = zeros16
            i1_v[pl.ds(pad - 16, 16)] = zeros16
        pltpu.sync_copy(i0_hbm.at[pl.ds(base, ew)], i0_v.at[pl.ds(0, ew)])
        pltpu.sync_copy(i1_hbm.at[pl.ds(base, ew)], i1_v.at[pl.ds(0, ew)])

        def body(k_it, _):
            off = k_it * 16
            g0 = plsc.load_gather(p_v, [i0_v[pl.ds(off, 16)]])
            g1 = plsc.load_gather(q_v, [i1_v[pl.ds(off, 16)]])
            out_v[pl.ds(off, 16)] = g0 + g1
            return _

        lax.fori_loop(0, steps, body, None)
        pltpu.sync_copy(out_v.at[pl.ds(0, ew)], out_hbm.at[pl.ds(base, ew)])

    return k(p, q, idx0, idx1)


def kernel(x, o_adj, s_adj, idx, Wo1, bo1, Wo2, bo2, Ws1, bs1, Ws2, bs2,
           ag1, ag2, Wd1, bd1, Wd2, bd2):
    n, nfeat = x.shape
    h1 = Wo1.shape[1]
    h2 = Wo2.shape[1]
    e_total = idx.shape[1]

    w = Wd1 @ Wd2
    wa = w[:h2, 0][None, :]
    wb = w[h2:, 0][None, :]
    c = (bd1 @ Wd2 + bd2).reshape(1, 1)

    bmp = 2000
    s_o, s_s = pl.pallas_call(
        _prep_body,
        grid=(pl.cdiv(n, bmp),),
        in_specs=[
            pl.BlockSpec((bmp, nfeat), lambda i: (i, 0)),
            pl.BlockSpec((nfeat, h1), lambda i: (0, 0)),
            pl.BlockSpec((nfeat, h1), lambda i: (0, 0)),
        ],
        out_specs=[
            pl.BlockSpec((bmp, h1), lambda i: (i, 0)),
            pl.BlockSpec((bmp, h1), lambda i: (i, 0)),
        ],
        out_shape=[
            jax.ShapeDtypeStruct((n, h1), jnp.float32),
            jax.ShapeDtypeStruct((n, h1), jnp.float32),
        ],
    )(x, Wo1, Ws1)

    bm = 128
    grid = (pl.cdiv(n, bm),)
    adj_spec = pl.BlockSpec((bm, n), lambda i: (i, 0))
    full = lambda r, c_: pl.BlockSpec((r, c_), lambda i: (0, 0))

    t_o, t_s = pl.pallas_call(
        _passA_body,
        grid=grid,
        in_specs=[
            adj_spec, adj_spec,
            full(n, h1), full(n, h1),
            full(1, h1), full(1, h1),
            full(h1, h2), full(h1, h2),
        ],
        out_specs=[
            pl.BlockSpec((bm, h2), lambda i: (i, 0)),
            pl.BlockSpec((bm, h2), lambda i: (i, 0)),
        ],
        out_shape=[
            jax.ShapeDtypeStruct((n, h2), jnp.float32),
            jax.ShapeDtypeStruct((n, h2), jnp.float32),
        ],
    )(o_adj, s_adj, s_o, s_s, bo1[None, :], bs1[None, :], Wo2, Ws2)

    pq = pl.pallas_call(
        _passB_body,
        grid=grid,
        in_specs=[
            adj_spec, adj_spec,
            full(n, h2), full(n, h2),
            full(1, h2), full(1, h2),
            full(1, h2), full(1, h2),
            full(1, h2), full(1, h2),
            full(1, 1),
        ],
        out_specs=pl.BlockSpec((bm, 8), lambda i: (i, 0)),
        out_shape=jax.ShapeDtypeStruct((n, 8), jnp.float32),
    )(o_adj, s_adj, t_o, t_s, bo2[None, :], bs2[None, :],
      ag1[None, :], ag2[None, :], wa, wb, c)

    p = pq[:, 0]
    q = pq[:, 1]

    info = plsc.get_sparse_core_info()
    nc, ns = info.num_cores, info.num_subcores
    out = _edge_gather(p, q, idx[0], idx[1], nc * ns, nc)
    return out[:, None]
